# Initial kernel scaffold; baseline (speedup 1.0000x reference)
import jax
import jax.numpy as jnp
from jax.experimental import pallas as pl


def kernel(x, edge_index, edge_attr, u, batch, W1, b1, W2, b2):
    raise NotImplementedError("write your pallas kernel here")


# speedup vs baseline: 4.1983x; 4.1983x over previous
import functools

import jax
import jax.numpy as jnp
from jax import lax
from jax.experimental import pallas as pl
from jax.experimental.pallas import tpu as pltpu
from jax.experimental.pallas import tpu_sc as plsc

N = 10000
D = 128
B = 256
G = 128
H = 128

NC = 2
NS = 16
NW = NC * NS
ROWS_W = N // NW
LAST_EXTRA = N - NW * ROWS_W
CHUNK = 128
NCHUNK = 3
STAGE = NCHUNK * CHUNK
ACC_ROWS = 272
DUMMY = 256


def _seg_kernel_body(x_hbm, ids_hbm, sums_hbm, cnts_hbm,
                     rows_v, ids_flat, ids2d, ones_v, zero_v,
                     acc_sh, cnt_sh):
    c = lax.axis_index("c")
    s = lax.axis_index("s")
    w = c * NS + s
    base = w * ROWS_W

    zf = jnp.zeros((16,), jnp.float32)
    for i in range(16):
        for j in range(D // 16):
            zero_v[i, pl.ds(j * 16, 16)] = zf
    one0 = jnp.where(lax.iota(jnp.int32, 16) == 0, 1.0, 0.0).astype(jnp.float32)
    for i in range(CHUNK):
        ones_v[i, pl.ds(0, 16)] = one0
    dvec = jnp.full((16,), DUMMY, jnp.int32)
    for k in range(STAGE // 16):
        ids_flat[pl.ds(k * 16, 16)] = dvec

    pltpu.sync_copy(zero_v, acc_sh.at[pl.ds(s * 16, 16)])
    pltpu.sync_copy(zero_v, cnt_sh.at[pl.ds(s * 16, 16)])

    @pl.when(w < NW - 1)
    def _():
        pltpu.sync_copy(ids_hbm.at[pl.ds(base, ROWS_W)],
                        ids_flat.at[pl.ds(0, ROWS_W)])
        pltpu.sync_copy(x_hbm.at[pl.ds(base, ROWS_W)],
                        rows_v.at[pl.ds(0, ROWS_W)])

    @pl.when(w == NW - 1)
    def _():
        pltpu.sync_copy(ids_hbm.at[pl.ds(base, ROWS_W + LAST_EXTRA)],
                        ids_flat.at[pl.ds(0, ROWS_W + LAST_EXTRA)])
        pltpu.sync_copy(x_hbm.at[pl.ds(base, ROWS_W + LAST_EXTRA)],
                        rows_v.at[pl.ds(0, ROWS_W + LAST_EXTRA)])

    for j in range(NCHUNK):
        for k in range(CHUNK // 16):
            ids2d[j, pl.ds(k * 16, 16)] = ids_flat[pl.ds(j * CHUNK + k * 16, 16)]

    plsc.subcore_barrier()

    for j in range(NCHUNK):
        pltpu.sync_copy(rows_v.at[pl.ds(j * CHUNK, CHUNK)],
                        acc_sh.at[ids2d.at[j]], add=True)
        pltpu.sync_copy(ones_v, cnt_sh.at[ids2d.at[j]], add=True)

    plsc.subcore_barrier()

    @pl.when(s == 0)
    def _():
        pltpu.sync_copy(acc_sh.at[pl.ds(0, B)], sums_hbm.at[c])
        pltpu.sync_copy(cnt_sh.at[pl.ds(0, B)], cnts_hbm.at[c])


_seg_reduce = pl.kernel(
    _seg_kernel_body,
    out_type=(jax.ShapeDtypeStruct((NC, B, D), jnp.float32),
              jax.ShapeDtypeStruct((NC, B, D), jnp.float32)),
    mesh=plsc.VectorSubcoreMesh(core_axis_name="c", subcore_axis_name="s"),
    scratch_types=[
        pltpu.VMEM((STAGE, D), jnp.float32),
        pltpu.VMEM((STAGE,), jnp.int32),
        pltpu.VMEM((NCHUNK, CHUNK), jnp.int32),
        pltpu.VMEM((CHUNK, D), jnp.float32),
        pltpu.VMEM((16, D), jnp.float32),
        pltpu.VMEM_SHARED((ACC_ROWS, D), jnp.float32),
        pltpu.VMEM_SHARED((ACC_ROWS, D), jnp.float32),
    ],
)


def _mlp_body(sums_ref, cnts_ref, u_ref, W1_ref, b1_ref, W2_ref, b2_ref,
              out_ref):
    seg = sums_ref[0] + sums_ref[1]
    cnt = cnts_ref[0, :, 0:1] + cnts_ref[1, :, 0:1]
    mean = seg / jnp.maximum(cnt, 1.0)
    u = u_ref[...]
    h = (lax.dot_general(u, W1_ref[0:G, :], (((1,), (0,)), ((), ())),
                         preferred_element_type=jnp.float32)
         + lax.dot_general(mean, W1_ref[G:G + D, :], (((1,), (0,)), ((), ())),
                           preferred_element_type=jnp.float32)
         + b1_ref[...])
    h = jnp.maximum(h, 0.0)
    out_ref[...] = (lax.dot_general(h, W2_ref[...], (((1,), (0,)), ((), ())),
                                    preferred_element_type=jnp.float32)
                    + b2_ref[...] + u)


def _mlp(sums, cnts, u, W1, b1, W2, b2):
    return pl.pallas_call(
        _mlp_body,
        out_shape=jax.ShapeDtypeStruct((B, G), jnp.float32),
    )(sums, cnts, u, W1, b1, W2, b2)


@jax.jit
def kernel(x, edge_index, edge_attr, u, batch, W1, b1, W2, b2):
    ids = batch.astype(jnp.int32)
    sums, cnts = _seg_reduce(x, ids)
    return _mlp(sums, cnts, u, W1, b1.reshape(1, H), W2, b2.reshape(1, G))


# speedup vs baseline: 5.0566x; 1.2044x over previous
import jax
import jax.numpy as jnp
from jax import lax
from jax.experimental import pallas as pl
from jax.experimental.pallas import tpu as pltpu
from jax.experimental.pallas import tpu_sc as plsc

N = 10000
D = 128
B = 256
G = 128
H = 128

NC = 2
NS = 16
NW = NC * NS
ROWS_W = N // NW
LAST_EXTRA = N - NW * ROWS_W
CHUNK = 128
NCHUNK = 3
STAGE = NCHUNK * CHUNK
ACC_ROWS = 272
DUMMY = 256

NPAD = 10240


def _seg_kernel_body(x_hbm, ids_hbm, sums_hbm,
                     rows_v, ids_flat, ids2d, zero_v, acc_sh,
                     sem_rows, sem_ids):
    c = lax.axis_index("c")
    s = lax.axis_index("s")
    w = c * NS + s
    base = w * ROWS_W

    @pl.when(w < NW - 1)
    def _():
        pltpu.async_copy(x_hbm.at[pl.ds(base, ROWS_W)],
                         rows_v.at[pl.ds(0, ROWS_W)], sem_rows)
        pltpu.async_copy(ids_hbm.at[pl.ds(base, ROWS_W)],
                         ids_flat.at[pl.ds(0, ROWS_W)], sem_ids)

    @pl.when(w == NW - 1)
    def _():
        pltpu.async_copy(x_hbm.at[pl.ds(base, ROWS_W + LAST_EXTRA)],
                         rows_v.at[pl.ds(0, ROWS_W + LAST_EXTRA)], sem_rows)
        pltpu.async_copy(ids_hbm.at[pl.ds(base, ROWS_W + LAST_EXTRA)],
                         ids_flat.at[pl.ds(0, ROWS_W + LAST_EXTRA)], sem_ids)

    zf = jnp.zeros((16,), jnp.float32)
    for i in range(16):
        for j in range(D // 16):
            zero_v[i, pl.ds(j * 16, 16)] = zf
    dvec = jnp.full((16,), DUMMY, jnp.int32)
    for j in range(NCHUNK):
        for k in range(CHUNK // 16):
            ids2d[j, pl.ds(k * 16, 16)] = dvec

    pltpu.sync_copy(zero_v, acc_sh.at[pl.ds(s * 16, 16)])

    lanei = lax.iota(jnp.int32, 16)

    def drain_and_repack(nrows):
        pltpu.make_async_copy(ids_hbm.at[pl.ds(0, nrows)],
                              ids_flat.at[pl.ds(0, nrows)], sem_ids).wait()
        nf = nrows // 16
        for k in range(nf):
            ids2d[k // 8, pl.ds((k % 8) * 16, 16)] = ids_flat[pl.ds(k * 16, 16)]
        rem = nrows - nf * 16
        if rem:
            tail = ids_flat[pl.ds(nf * 16, 16)]
            tail = jnp.where(lanei < rem, tail, DUMMY)
            ids2d[nf // 8, pl.ds((nf % 8) * 16, 16)] = tail
        pltpu.make_async_copy(x_hbm.at[pl.ds(0, nrows)],
                              rows_v.at[pl.ds(0, nrows)], sem_rows).wait()

    @pl.when(w < NW - 1)
    def _():
        drain_and_repack(ROWS_W)

    @pl.when(w == NW - 1)
    def _():
        drain_and_repack(ROWS_W + LAST_EXTRA)

    plsc.subcore_barrier()

    for j in range(NCHUNK):
        pltpu.sync_copy(rows_v.at[pl.ds(j * CHUNK, CHUNK)],
                        acc_sh.at[ids2d.at[j]], add=True)

    plsc.subcore_barrier()

    @pl.when(s == 0)
    def _():
        pltpu.sync_copy(acc_sh.at[pl.ds(0, B)], sums_hbm.at[c])


_seg_reduce = pl.kernel(
    _seg_kernel_body,
    out_type=jax.ShapeDtypeStruct((NC, B, D), jnp.float32),
    mesh=plsc.VectorSubcoreMesh(core_axis_name="c", subcore_axis_name="s"),
    scratch_types=[
        pltpu.VMEM((STAGE, D), jnp.float32),
        pltpu.VMEM((STAGE,), jnp.int32),
        pltpu.VMEM((NCHUNK, CHUNK), jnp.int32),
        pltpu.VMEM((16, D), jnp.float32),
        pltpu.VMEM_SHARED((ACC_ROWS, D), jnp.float32),
        pltpu.SemaphoreType.DMA,
        pltpu.SemaphoreType.DMA,
    ],
)


def _hist_body(ids_ref, cnt_ref):
    segs = lax.broadcasted_iota(jnp.int32, (B, 1), 0)

    def step(r, acc):
        row = ids_ref[pl.ds(r, 1), :]
        return acc + (row == segs).astype(jnp.float32)

    acc = lax.fori_loop(0, NPAD // 128, step, jnp.zeros((B, 128), jnp.float32))
    cnt_ref[...] = jnp.sum(acc, axis=1, keepdims=True)


def _hist(ids_pad):
    return pl.pallas_call(
        _hist_body,
        out_shape=jax.ShapeDtypeStruct((B, 1), jnp.float32),
    )(ids_pad)


def _mlp_body(sums_ref, cnt_ref, u_ref, W1_ref, b1_ref, W2_ref, b2_ref,
              out_ref):
    seg = sums_ref[0] + sums_ref[1]
    cnt = cnt_ref[...]
    mean = seg / jnp.maximum(cnt, 1.0)
    u = u_ref[...]
    h = (lax.dot_general(u, W1_ref[0:G, :], (((1,), (0,)), ((), ())),
                         preferred_element_type=jnp.float32)
         + lax.dot_general(mean, W1_ref[G:G + D, :], (((1,), (0,)), ((), ())),
                           preferred_element_type=jnp.float32)
         + b1_ref[...])
    h = jnp.maximum(h, 0.0)
    out_ref[...] = (lax.dot_general(h, W2_ref[...], (((1,), (0,)), ((), ())),
                                    preferred_element_type=jnp.float32)
                    + b2_ref[...] + u)


def _mlp(sums, cnt, u, W1, b1, W2, b2):
    return pl.pallas_call(
        _mlp_body,
        out_shape=jax.ShapeDtypeStruct((B, G), jnp.float32),
    )(sums, cnt, u, W1, b1, W2, b2)


@jax.jit
def kernel(x, edge_index, edge_attr, u, batch, W1, b1, W2, b2):
    ids = batch.astype(jnp.int32)
    ids_pad = jnp.concatenate(
        [ids, jnp.full((NPAD - N,), B, jnp.int32)]).reshape(NPAD // 128, 128)
    cnt = _hist(ids_pad)
    sums = _seg_reduce(x, ids)
    return _mlp(sums, cnt, u, W1, b1.reshape(1, H), W2, b2.reshape(1, G))
